# baseline (speedup 1.0000x reference)
import jax
import jax.numpy as jnp
from jax.experimental import pallas as pl
from jax.experimental.pallas import tpu as pltpu

_ROWS, _COLS = 128, 100000
_BLOCK_ROWS = 8
_GRID = _ROWS // _BLOCK_ROWS

_K0 = 0
_K1 = 42
_K2 = _K0 ^ _K1 ^ 0x1BD11BDA

_ROT_A = (13, 15, 26, 6)
_ROT_B = (17, 29, 16, 24)
_INJECT = (
    (_K1, (_K2 + 1) & 0xFFFFFFFF),
    (_K2, (_K0 + 2) & 0xFFFFFFFF),
    (_K0, (_K1 + 3) & 0xFFFFFFFF),
    (_K1, (_K2 + 4) & 0xFFFFFFFF),
    (_K2, (_K0 + 5) & 0xFFFFFFFF),
)


def _threefry2x32(x0, x1):
    def rotl(v, r):
        return (v << jnp.uint32(r)) | (v >> jnp.uint32(32 - r))

    x0 = x0 + jnp.uint32(_K0)
    x1 = x1 + jnp.uint32(_K1)
    for rots, (i0, i1) in zip((_ROT_A, _ROT_B, _ROT_A, _ROT_B, _ROT_A), _INJECT):
        for r in rots:
            x0 = x0 + x1
            x1 = rotl(x1, r) ^ x0
        x0 = x0 + jnp.uint32(i0)
        x1 = x1 + jnp.uint32(i1)
    return x0, x1


def _gumbel_softmax_block(x_ref, o_ref):
    i = pl.program_id(0)
    shape = (_BLOCK_ROWS, _COLS)
    base = (i * (_BLOCK_ROWS * _COLS)).astype(jnp.uint32)
    sub = jax.lax.broadcasted_iota(jnp.uint32, shape, 0) * jnp.uint32(_COLS)
    lane = jax.lax.broadcasted_iota(jnp.uint32, shape, 1)
    idx = base + sub + lane

    b0, b1 = _threefry2x32(jnp.zeros(shape, jnp.uint32), idx)
    bits = b0 ^ b1
    fbits = (bits >> jnp.uint32(9)) | jnp.uint32(0x3F800000)
    u = jax.lax.bitcast_convert_type(fbits, jnp.float32) - jnp.float32(1.0)
    g = -jnp.log(-jnp.log(u + jnp.float32(1e-10)) + jnp.float32(1e-10))

    y = x_ref[...] + g
    m = jnp.max(y, axis=-1, keepdims=True)
    e = jnp.exp(y - m)
    o_ref[...] = e / jnp.sum(e, axis=-1, keepdims=True)


def kernel(logits):
    return pl.pallas_call(
        _gumbel_softmax_block,
        grid=(_GRID,),
        in_specs=[pl.BlockSpec((_BLOCK_ROWS, _COLS), lambda i: (i, 0))],
        out_specs=pl.BlockSpec((_BLOCK_ROWS, _COLS), lambda i: (i, 0)),
        out_shape=jax.ShapeDtypeStruct((_ROWS, _COLS), jnp.float32),
        compiler_params=pltpu.CompilerParams(
            dimension_semantics=("arbitrary",),
        ),
    )(logits)
